# Initial kernel scaffold; baseline (speedup 1.0000x reference)
import functools

import jax
import jax.numpy as jnp
from jax import lax
from jax.experimental import pallas as pl
from jax.experimental.pallas import tpu as pltpu
from jax.experimental.pallas import tpu_sc as plsc

V = 10000
E = 160000
B = 2
N = B * V
R = 2500
GRID = N // R
BLK_PER_BATCH = V // R

NS = 16
VPT = V // NS
EPT = E // NS

_MESH = dict(core_axis_name="c", subcore_axis_name="s")


def _zero_fill(ref, rows, cols):
    zv = jnp.zeros((16,), jnp.float32)

    def body(r, _):
        for j in range(cols // 16):
            ref[r, pl.ds(j * 16, 16)] = zv
        return 0

    lax.fori_loop(0, rows, body, 0)


def _make_deg_kernel():
    DCH = 1000
    NCHUNK = EPT // DCH

    @functools.partial(
        pl.kernel,
        out_type=jax.ShapeDtypeStruct((2 * V, 16), jnp.float32),
        mesh=plsc.VectorSubcoreMesh(**_MESH),
        scratch_types=[
            pltpu.VMEM((DCH,), jnp.int32),
            pltpu.VMEM((DCH, 16), jnp.float32),
            pltpu.VMEM((VPT, 16), jnp.float32),
            pltpu.VMEM_SHARED((V, 16), jnp.float32),
        ],
    )
    def deg_kernel(ei_hbm, out_hbm, idx_v, ones_v, zero_v, acc_sh):
        c = lax.axis_index("c")
        s = lax.axis_index("s")

        lane = lax.iota(jnp.int32, 16)
        one_hot = jnp.where(lane == 0, 1.0, 0.0).astype(jnp.float32)

        def fill_ones(r, _):
            ones_v[r, :] = one_hot
            return 0

        lax.fori_loop(0, DCH, fill_ones, 0)
        _zero_fill(zero_v, VPT, 16)
        pltpu.sync_copy(zero_v, acc_sh.at[pl.ds(s * VPT, VPT)])
        plsc.subcore_barrier()

        def chunk(k, _):
            base = s * EPT + k * DCH
            pltpu.sync_copy(ei_hbm.at[c, pl.ds(base, DCH)], idx_v)
            pltpu.sync_copy(ones_v, acc_sh.at[idx_v], add=True)
            return 0

        lax.fori_loop(0, NCHUNK, chunk, 0)
        plsc.subcore_barrier()
        pltpu.sync_copy(acc_sh.at[pl.ds(s * VPT, VPT)],
                        out_hbm.at[pl.ds(c * V + s * VPT, VPT)])

    return deg_kernel


def _make_g_kernel(C, ECH):
    NCHUNK = EPT // ECH

    @functools.partial(
        pl.kernel,
        out_type=jax.ShapeDtypeStruct((2 * V, C), jnp.float32),
        mesh=plsc.VectorSubcoreMesh(**_MESH),
        scratch_types=[
            pltpu.VMEM((ECH,), jnp.int32),
            pltpu.VMEM((ECH,), jnp.int32),
            pltpu.VMEM((ECH, C), jnp.float32),
            pltpu.VMEM_SHARED((V, C), jnp.float32),
            pltpu.SemaphoreType.DMA,
        ],
    )
    def g_kernel(x_hbm, ei_hbm, out_hbm, sidx_v, didx_v, rows_v, acc_sh, sem):
        c = lax.axis_index("c")
        s = lax.axis_index("s")
        off = c * V

        _zero_fill(rows_v, ECH, C)
        n_full = VPT // ECH
        for j in range(n_full):
            pltpu.sync_copy(rows_v, acc_sh.at[pl.ds(s * VPT + j * ECH, ECH)])
        rem = VPT - n_full * ECH
        if rem:
            pltpu.sync_copy(rows_v.at[pl.ds(0, rem)],
                            acc_sh.at[pl.ds(s * VPT + n_full * ECH, rem)])
        plsc.subcore_barrier()

        def chunk(k, _):
            base = s * EPT + k * ECH
            pltpu.sync_copy(ei_hbm.at[0, pl.ds(base, ECH)], sidx_v)
            pltpu.sync_copy(ei_hbm.at[1, pl.ds(base, ECH)], didx_v)

            def shift(j, _):
                sl = pl.ds(j * 16, 16)
                sidx_v[sl] = sidx_v[sl] + off
                return 0

            lax.fori_loop(0, ECH // 16, shift, 0)
            pltpu.async_copy(x_hbm.at[sidx_v], rows_v, sem).wait()
            pltpu.sync_copy(rows_v, acc_sh.at[didx_v], add=True)
            return 0

        lax.fori_loop(0, NCHUNK, chunk, 0)
        plsc.subcore_barrier()
        pltpu.sync_copy(acc_sh.at[pl.ds(s * VPT, VPT)],
                        out_hbm.at[pl.ds(c * V + s * VPT, VPT)])

    return g_kernel


def _row_spec(c):
    return pl.BlockSpec((R, c), lambda i: (i, 0))


def _vrow_spec():
    return pl.BlockSpec((R, 1), lambda i: (i % BLK_PER_BATCH, 0))


def _full_spec(shape):
    nd = len(shape)
    return pl.BlockSpec(shape, lambda i, _n=nd: (0,) * _n)


def _prep_body(ds_ref, dd_ref, a_ref, c_ref, m_ref):
    a = lax.rsqrt(jnp.maximum(ds_ref[...], 1.0))
    c = lax.rsqrt(jnp.maximum(dd_ref[...], 1.0))
    a_ref[...] = a
    c_ref[...] = c
    m_ref[...] = -(a * c)


def _prep(deg_s, deg_d):
    out = jax.ShapeDtypeStruct((N, 1), jnp.float32)
    return pl.pallas_call(
        _prep_body,
        grid=(GRID,),
        in_specs=[_vrow_spec(), _vrow_spec()],
        out_specs=[pl.BlockSpec((R, 1), lambda i: (i, 0))] * 3,
        out_shape=[out, out, out],
    )(deg_s, deg_d)


def _scale_body(x_ref, s_ref, y_ref):
    y_ref[...] = x_ref[...] * s_ref[...]


def _scale(x, s, c):
    return pl.pallas_call(
        _scale_body,
        grid=(GRID,),
        in_specs=[_row_spec(c), pl.BlockSpec((R, 1), lambda i: (i, 0))],
        out_specs=_row_spec(c),
        out_shape=jax.ShapeDtypeStruct((N, c), jnp.float32),
    )(x, s)


def _combine_body(relu, residual, *refs):
    if residual:
        h_ref, g1_ref, g2_ref, c_ref, w_ref, b_ref, r_ref, o_ref = refs
    else:
        h_ref, g1_ref, g2_ref, c_ref, w_ref, b_ref, o_ref = refs
    c = c_ref[...]
    h = h_ref[...]
    t1 = -(c * g1_ref[...])
    t2 = -2.0 * (c * g2_ref[...])
    acc = jnp.dot(h, w_ref[0] - w_ref[2], preferred_element_type=jnp.float32)
    acc = acc + jnp.dot(t1, w_ref[1], preferred_element_type=jnp.float32)
    acc = acc + jnp.dot(t2, w_ref[2], preferred_element_type=jnp.float32)
    acc = acc + b_ref[...]
    if residual:
        acc = acc + r_ref[...]
    if relu:
        acc = jnp.maximum(acc, 0.0)
    o_ref[...] = acc


def _combine(h, g1, g2, c_row, W, b, cin, cout, relu, resid=None):
    ins = [h, g1, g2, c_row, W, b.reshape(1, cout)]
    specs = [_row_spec(cin), _row_spec(cin), _row_spec(cin),
             pl.BlockSpec((R, 1), lambda i: (i, 0)),
             _full_spec((3, cin, cout)), _full_spec((1, cout))]
    if resid is not None:
        ins.append(resid)
        specs.append(_row_spec(cout))
    return pl.pallas_call(
        functools.partial(_combine_body, relu, resid is not None),
        grid=(GRID,),
        in_specs=specs,
        out_specs=_row_spec(cout),
        out_shape=jax.ShapeDtypeStruct((N, cout), jnp.float32),
    )(*ins)


def _stats_body(x_ref, s_ref, q_ref):
    @pl.when(pl.program_id(0) == 0)
    def _():
        s_ref[...] = jnp.zeros_like(s_ref)
        q_ref[...] = jnp.zeros_like(q_ref)

    x = x_ref[...]
    s_ref[...] += jnp.sum(x, axis=0, keepdims=True)
    q_ref[...] += jnp.sum(x * x, axis=0, keepdims=True)


def _bn_stats(x, c):
    out = jax.ShapeDtypeStruct((1, c), jnp.float32)
    return pl.pallas_call(
        _stats_body,
        grid=(GRID,),
        in_specs=[_row_spec(c)],
        out_specs=[_full_spec((1, c))] * 2,
        out_shape=[out, out],
    )(x)


def _bn_apply_body(x_ref, s_ref, q_ref, g_ref, be_ref, a_ref, h_ref, u_ref):
    mean = s_ref[...] * (1.0 / N)
    var = q_ref[...] * (1.0 / N) - mean * mean
    scale = g_ref[...] * lax.rsqrt(var + 1e-5)
    shift = be_ref[...] - mean * scale
    h = x_ref[...] * scale + shift
    h_ref[...] = h
    u_ref[...] = h * a_ref[...]


def _bn_apply(x, sums, sq, g, be, a_row, c):
    out = jax.ShapeDtypeStruct((N, c), jnp.float32)
    return pl.pallas_call(
        _bn_apply_body,
        grid=(GRID,),
        in_specs=[_row_spec(c), _full_spec((1, c)), _full_spec((1, c)),
                  _full_spec((1, c)), _full_spec((1, c)),
                  pl.BlockSpec((R, 1), lambda i: (i, 0))],
        out_specs=[_row_spec(c)] * 2,
        out_shape=[out, out],
    )(x, sums, sq, g.reshape(1, c), be.reshape(1, c), a_row)


def _pool_body(y_ref, o_ref):
    i = pl.program_id(0)

    @pl.when(i % BLK_PER_BATCH == 0)
    def _():
        o_ref[...] = jnp.full_like(o_ref, -jnp.inf)

    o_ref[...] = jnp.maximum(o_ref[...],
                             jnp.max(y_ref[...], axis=0, keepdims=True))

    @pl.when(i % BLK_PER_BATCH == BLK_PER_BATCH - 1)
    def _():
        v = o_ref[...]
        m = jnp.max(v, axis=1, keepdims=True)
        e = jnp.exp(v - m)
        o_ref[...] = v - m - jnp.log(jnp.sum(e, axis=1, keepdims=True))


def _pool(y, cout):
    return pl.pallas_call(
        _pool_body,
        grid=(GRID,),
        in_specs=[_row_spec(cout)],
        out_specs=pl.BlockSpec((1, cout), lambda i: (i // BLK_PER_BATCH, 0)),
        out_shape=jax.ShapeDtypeStruct((B, cout), jnp.float32),
    )(y)


_G128 = _make_g_kernel(128, 400)
_G64 = _make_g_kernel(64, 1000)
_DEG = _make_deg_kernel()


def _cheb(h, u0, ei, c_row, m_row, W, b, cin, cout, g_call, relu,
          resid=None):
    g1 = g_call(u0, ei)
    u1 = _scale(g1, m_row, cin)
    g2 = g_call(u1, ei)
    return _combine(h, g1, g2, c_row, W, b, cin, cout, relu, resid)


def kernel(x, edge_index, W_in, b_in, g1, be1, W_r1, b_r1, g2, be2, W_r2,
           b_r2, g_out, be_out, W_out, b_out):
    ei = edge_index.astype(jnp.int32)
    xt = jnp.transpose(x, (0, 2, 1)).reshape(N, -1)
    cin = xt.shape[1]

    deg16 = _DEG(ei)
    deg_s = deg16[:V, 0:1]
    deg_d = deg16[V:, 0:1]
    a_row, c_row, m_row = _prep(deg_s, deg_d)

    u0 = _scale(xt, a_row, cin)
    h0 = _cheb(xt, u0, ei, c_row, m_row, W_in, b_in, cin, 64, _G128,
               relu=True)

    s1, q1 = _bn_stats(h0, 64)
    hb, u0 = _bn_apply(h0, s1, q1, g1, be1, a_row, 64)
    o = _cheb(hb, u0, ei, c_row, m_row, W_r1, b_r1, 64, 64, _G64, relu=True)
    s2, q2 = _bn_stats(o, 64)
    ob, u0 = _bn_apply(o, s2, q2, g2, be2, a_row, 64)
    out = _cheb(ob, u0, ei, c_row, m_row, W_r2, b_r2, 64, 64, _G64,
                relu=True, resid=hb)

    s3, q3 = _bn_stats(out, 64)
    z, u0 = _bn_apply(out, s3, q3, g_out, be_out, a_row, 64)
    y = _cheb(z, u0, ei, c_row, m_row, W_out, b_out, 64, 10, _G64, relu=True)
    return _pool(y, 10)


# speedup vs baseline: 3.6915x; 3.6915x over previous
import functools

import jax
import jax.numpy as jnp
from jax import lax
from jax.experimental import pallas as pl
from jax.experimental.pallas import tpu as pltpu
from jax.experimental.pallas import tpu_sc as plsc

V = 10000
E = 160000
B = 2
N = B * V
R = 2000
GRID = N // R
BLK_PER_BATCH = V // R

NS = 16
VPT = V // NS
EPT = E // NS

_MESH = dict(core_axis_name="c", subcore_axis_name="s")


def _zero_fill(ref, rows, cols):
    zv = jnp.zeros((16,), jnp.float32)

    def body(r, _):
        for j in range(cols // 16):
            ref[r, pl.ds(j * 16, 16)] = zv
        return 0

    lax.fori_loop(0, rows, body, 0)


def _make_g_kernel(C, ECH):
    NCHUNK = EPT // ECH

    @functools.partial(
        pl.kernel,
        out_type=jax.ShapeDtypeStruct((2, NS, VPT, C), jnp.float32),
        mesh=plsc.VectorSubcoreMesh(**_MESH),
        scratch_types=[
            pltpu.VMEM((EPT,), jnp.int32),
            pltpu.VMEM((ECH,), jnp.int32),
            pltpu.VMEM((ECH, C), jnp.float32),
            pltpu.VMEM_SHARED((V, C), jnp.float32),
            pltpu.SemaphoreType.DMA,
        ],
    )
    def g_kernel(x_hbm, src_hbm, dst_hbm, out_hbm, sidx_v, didx_v, rows_v,
                 acc_sh, sem):
        c = lax.axis_index("c")
        s = lax.axis_index("s")
        off = c * V

        tbase = pl.multiple_of(s * EPT, 8)
        pltpu.sync_copy(src_hbm.at[pl.ds(tbase, EPT)], sidx_v)

        def shift(j, _):
            sl = pl.ds(j * 16, 16)
            sidx_v[sl] = sidx_v[sl] + off
            return 0

        lax.fori_loop(0, EPT // 16, shift, 0)

        _zero_fill(rows_v, ECH, C)
        n_full = VPT // ECH
        rem = VPT - n_full * ECH
        for j in range(n_full):
            pltpu.sync_copy(rows_v, acc_sh.at[pl.ds(s * VPT + j * ECH, ECH)])
        if rem:
            pltpu.sync_copy(rows_v.at[pl.ds(0, rem)],
                            acc_sh.at[pl.ds(s * VPT + n_full * ECH, rem)])
        plsc.subcore_barrier()

        def chunk(k, _):
            base = pl.multiple_of(s * EPT + k * ECH, 8)
            pltpu.sync_copy(dst_hbm.at[pl.ds(base, ECH)], didx_v)
            kb = pl.multiple_of(k * ECH, 8)
            pltpu.async_copy(x_hbm.at[sidx_v.at[pl.ds(kb, ECH)]], rows_v,
                             sem).wait()
            pltpu.sync_copy(rows_v, acc_sh.at[didx_v], add=True)
            return 0

        lax.fori_loop(0, NCHUNK, chunk, 0)
        plsc.subcore_barrier()
        for j in range(n_full):
            pltpu.sync_copy(acc_sh.at[pl.ds(s * VPT + j * ECH, ECH)], rows_v)
            pltpu.sync_copy(rows_v, out_hbm.at[c, s, pl.ds(j * ECH, ECH)])
        if rem:
            pltpu.sync_copy(acc_sh.at[pl.ds(s * VPT + n_full * ECH, rem)],
                            rows_v.at[pl.ds(0, rem)])
            pltpu.sync_copy(rows_v.at[pl.ds(0, rem)],
                            out_hbm.at[c, s, pl.ds(n_full * ECH, rem)])

    return g_kernel


def _row_spec(c):
    return pl.BlockSpec((R, c), lambda i: (i, 0))


def _col_spec():
    return pl.BlockSpec((R, 1), lambda i: (i, 0))


def _vrow_spec():
    return pl.BlockSpec((R, 1), lambda i: (i % BLK_PER_BATCH, 0))


def _full_spec(shape):
    nd = len(shape)
    return pl.BlockSpec(shape, lambda i, _n=nd: (0,) * _n)


def _prep_body(ds_ref, dd_ref, a_ref, c_ref, m_ref):
    a = 1.0 / jnp.sqrt(jnp.maximum(ds_ref[...], 1.0))
    c = 1.0 / jnp.sqrt(jnp.maximum(dd_ref[...], 1.0))
    a_ref[...] = a
    c_ref[...] = c
    m_ref[...] = -(a * c)


def _prep(deg_s, deg_d):
    out = jax.ShapeDtypeStruct((N, 1), jnp.float32)
    return pl.pallas_call(
        _prep_body,
        grid=(GRID,),
        in_specs=[_vrow_spec(), _vrow_spec()],
        out_specs=[_col_spec()] * 3,
        out_shape=[out, out, out],
    )(deg_s, deg_d)


def _scale_body(x_ref, s_ref, y_ref):
    y_ref[...] = x_ref[...] * s_ref[...]


def _scale(x, s, c):
    return pl.pallas_call(
        _scale_body,
        grid=(GRID,),
        in_specs=[_row_spec(c), _col_spec()],
        out_specs=_row_spec(c),
        out_shape=jax.ShapeDtypeStruct((N, c), jnp.float32),
    )(x, s)


def _scale_pad_body(x_ref, s_ref, y_ref):
    u = x_ref[...] * s_ref[...]
    y_ref[...] = jnp.concatenate([u, jnp.zeros_like(u)], axis=1)


def _scale_pad(x, s):
    return pl.pallas_call(
        _scale_pad_body,
        grid=(GRID,),
        in_specs=[_row_spec(64), _col_spec()],
        out_specs=_row_spec(128),
        out_shape=jax.ShapeDtypeStruct((N, 128), jnp.float32),
    )(x, s)


def _scale_split_body(x_ref, s_ref, ya_ref, yb_ref):
    u = x_ref[...] * s_ref[...]
    z = jnp.zeros((u.shape[0], 64), jnp.float32)
    ya_ref[...] = jnp.concatenate([u[:, :64], z], axis=1)
    yb_ref[...] = jnp.concatenate([u[:, 64:], z], axis=1)


def _scale_split(x, s):
    out = jax.ShapeDtypeStruct((N, 128), jnp.float32)
    return pl.pallas_call(
        _scale_split_body,
        grid=(GRID,),
        in_specs=[_row_spec(128), _col_spec()],
        out_specs=[_row_spec(128)] * 2,
        out_shape=[out, out],
    )(x, s)


def _combine_in_body(h_ref, g1a_ref, g1b_ref, g2a_ref, g2b_ref, c_ref,
                     w_ref, b_ref, o_ref):
    c = c_ref[...]
    w1 = w_ref[1]
    w2 = w_ref[2]
    acc = jnp.dot(h_ref[...], w_ref[0] - w2,
                  preferred_element_type=jnp.float32,
                  precision=lax.Precision.HIGHEST)
    acc = acc + jnp.dot(-(c * g1a_ref[...][:, :64]), w1[:64],
                        preferred_element_type=jnp.float32,
                  precision=lax.Precision.HIGHEST)
    acc = acc + jnp.dot(-(c * g1b_ref[...][:, :64]), w1[64:],
                        preferred_element_type=jnp.float32,
                  precision=lax.Precision.HIGHEST)
    acc = acc + jnp.dot(-2.0 * (c * g2a_ref[...][:, :64]), w2[:64],
                        preferred_element_type=jnp.float32,
                  precision=lax.Precision.HIGHEST)
    acc = acc + jnp.dot(-2.0 * (c * g2b_ref[...][:, :64]), w2[64:],
                        preferred_element_type=jnp.float32,
                  precision=lax.Precision.HIGHEST)
    acc = acc + b_ref[...]
    o_ref[...] = jnp.maximum(acc, 0.0)


def _combine_in(h, g1a, g1b, g2a, g2b, c_row, W, b):
    return pl.pallas_call(
        _combine_in_body,
        grid=(GRID,),
        in_specs=[_row_spec(128)] * 5 +
                 [_col_spec(), _full_spec((3, 128, 64)), _full_spec((1, 64))],
        out_specs=_row_spec(64),
        out_shape=jax.ShapeDtypeStruct((N, 64), jnp.float32),
    )(h, g1a, g1b, g2a, g2b, c_row, W, b.reshape(1, 64))


def _combine_body(relu, residual, *refs):
    if residual:
        h_ref, g1_ref, g2_ref, c_ref, w_ref, b_ref, r_ref, o_ref = refs
    else:
        h_ref, g1_ref, g2_ref, c_ref, w_ref, b_ref, o_ref = refs
    c = c_ref[...]
    t1 = -(c * g1_ref[...][:, :64])
    t2 = -2.0 * (c * g2_ref[...][:, :64])
    acc = jnp.dot(h_ref[...], w_ref[0] - w_ref[2],
                  preferred_element_type=jnp.float32,
                  precision=lax.Precision.HIGHEST)
    acc = acc + jnp.dot(t1, w_ref[1], preferred_element_type=jnp.float32,
                  precision=lax.Precision.HIGHEST)
    acc = acc + jnp.dot(t2, w_ref[2], preferred_element_type=jnp.float32,
                  precision=lax.Precision.HIGHEST)
    acc = acc + b_ref[...]
    if residual:
        acc = acc + r_ref[...]
    if relu:
        acc = jnp.maximum(acc, 0.0)
    o_ref[...] = acc


def _combine(h, g1, g2, c_row, W, b, cout, relu, resid=None):
    ins = [h, g1, g2, c_row, W, b.reshape(1, cout)]
    specs = [_row_spec(64), _row_spec(128), _row_spec(128), _col_spec(),
             _full_spec((3, 64, cout)), _full_spec((1, cout))]
    if resid is not None:
        ins.append(resid)
        specs.append(_row_spec(cout))
    return pl.pallas_call(
        functools.partial(_combine_body, relu, resid is not None),
        grid=(GRID,),
        in_specs=specs,
        out_specs=_row_spec(cout),
        out_shape=jax.ShapeDtypeStruct((N, cout), jnp.float32),
    )(*ins)


def _stats_body(x_ref, s_ref, q_ref):
    @pl.when(pl.program_id(0) == 0)
    def _():
        s_ref[...] = jnp.zeros_like(s_ref)
        q_ref[...] = jnp.zeros_like(q_ref)

    x = x_ref[...]
    s_ref[...] += jnp.sum(x, axis=0, keepdims=True)
    q_ref[...] += jnp.sum(x * x, axis=0, keepdims=True)


def _bn_stats(x, c):
    out = jax.ShapeDtypeStruct((1, c), jnp.float32)
    return pl.pallas_call(
        _stats_body,
        grid=(GRID,),
        in_specs=[_row_spec(c)],
        out_specs=[_full_spec((1, c))] * 2,
        out_shape=[out, out],
    )(x)


def _bn_apply_body(x_ref, s_ref, q_ref, g_ref, be_ref, a_ref, h_ref, u_ref):
    mean = s_ref[...] * (1.0 / N)
    var = q_ref[...] * (1.0 / N) - mean * mean
    scale = g_ref[...] / jnp.sqrt(var + 1e-5)
    shift = be_ref[...] - mean * scale
    h = x_ref[...] * scale + shift
    h_ref[...] = h
    u = h * a_ref[...]
    u_ref[...] = jnp.concatenate([u, jnp.zeros_like(u)], axis=1)


def _bn_apply(x, sums, sq, g, be, a_row):
    return pl.pallas_call(
        _bn_apply_body,
        grid=(GRID,),
        in_specs=[_row_spec(64)] + [_full_spec((1, 64))] * 4 + [_col_spec()],
        out_specs=[_row_spec(64), _row_spec(128)],
        out_shape=[jax.ShapeDtypeStruct((N, 64), jnp.float32),
                   jax.ShapeDtypeStruct((N, 128), jnp.float32)],
    )(x, sums, sq, g.reshape(1, 64), be.reshape(1, 64), a_row)


def _pool_body(y_ref, o_ref):
    i = pl.program_id(0)

    @pl.when(i == 0)
    def _():
        o_ref[...] = jnp.full_like(o_ref, -jnp.inf)

    b = i // BLK_PER_BATCH
    row = lax.broadcasted_iota(jnp.int32, (B, o_ref.shape[1]), 0)
    blk = jnp.max(y_ref[...], axis=0, keepdims=True)
    cur = o_ref[...]
    o_ref[...] = jnp.where(row == b, jnp.maximum(cur, blk), cur)

    @pl.when(i == GRID - 1)
    def _():
        v = o_ref[...]
        m = jnp.max(v, axis=1, keepdims=True)
        e = jnp.exp(v - m)
        o_ref[...] = v - m - jnp.log(jnp.sum(e, axis=1, keepdims=True))


def _pool(y, cout):
    return pl.pallas_call(
        _pool_body,
        grid=(GRID,),
        in_specs=[_row_spec(cout)],
        out_specs=pl.BlockSpec((B, cout), lambda i: (0, 0)),
        out_shape=jax.ShapeDtypeStruct((B, cout), jnp.float32),
    )(y)


_G128_RAW = _make_g_kernel(128, 200)


def _G(xf, src, dst):
    return _G128_RAW(xf, src, dst).reshape(N, 128)


def _cheb64(h, u0, src, dst, c_row, m_row, W, b, cout, relu, resid=None):
    g1 = _G(u0, src, dst)
    u1 = _scale(g1, m_row, 128)
    g2 = _G(u1, src, dst)
    return _combine(h, g1, g2, c_row, W, b, cout, relu, resid)


def kernel(x, edge_index, W_in, b_in, g1, be1, W_r1, b_r1, g2, be2, W_r2,
           b_r2, g_out, be_out, W_out, b_out):
    ei = edge_index.astype(jnp.int32)
    src = ei[0]
    dst = ei[1]
    xt = jnp.transpose(x, (0, 2, 1)).reshape(N, -1)

    ones_t = jnp.ones((N, 128), jnp.float32)
    deg_d = _G(ones_t, src, dst)[:V, 0:1]
    deg_s = _G(ones_t, dst, src)[:V, 0:1]
    a_row, c_row, m_row = _prep(deg_s, deg_d)

    u0a, u0b = _scale_split(xt, a_row)
    g1a = _G(u0a, src, dst)
    g1b = _G(u0b, src, dst)
    u1a = _scale(g1a, m_row, 128)
    u1b = _scale(g1b, m_row, 128)
    g2a = _G(u1a, src, dst)
    g2b = _G(u1b, src, dst)
    h0 = _combine_in(xt, g1a, g1b, g2a, g2b, c_row, W_in, b_in)

    s1, q1 = _bn_stats(h0, 64)
    hb, u0 = _bn_apply(h0, s1, q1, g1, be1, a_row)
    o = _cheb64(hb, u0, src, dst, c_row, m_row, W_r1, b_r1, 64, relu=True)
    s2, q2 = _bn_stats(o, 64)
    ob, u0 = _bn_apply(o, s2, q2, g2, be2, a_row)
    out = _cheb64(ob, u0, src, dst, c_row, m_row, W_r2, b_r2, 64,
                  relu=True, resid=hb)

    s3, q3 = _bn_stats(out, 64)
    z, u0 = _bn_apply(out, s3, q3, g_out, be_out, a_row)
    y = _cheb64(z, u0, src, dst, c_row, m_row, W_out, b_out, 10, relu=True)
    return _pool(y, 10)


# speedup vs baseline: 5.6852x; 1.5401x over previous
import functools

import jax
import jax.numpy as jnp
from jax import lax
from jax.experimental import pallas as pl
from jax.experimental.pallas import tpu as pltpu
from jax.experimental.pallas import tpu_sc as plsc

V = 10000
E = 160000
B = 2
RV = 2000
GRID = V // RV

NS = 16
VPT = V // NS
EPC = E // 2
EPT = EPC // NS
ECH = 200

_MESH = dict(core_axis_name="c", subcore_axis_name="s")
_HI = lax.Precision.HIGHEST


def _zero_fill(ref, rows, cols):
    zv = jnp.zeros((16,), jnp.float32)

    def body(r, _):
        for j in range(cols // 16):
            ref[r, pl.ds(j * 16, 16)] = zv
        return 0

    lax.fori_loop(0, rows, body, 0)


def _make_g_kernel():
    NCHUNK = EPT // ECH

    @functools.partial(
        pl.kernel,
        out_type=jax.ShapeDtypeStruct((2, NS, VPT, 128), jnp.float32),
        mesh=plsc.VectorSubcoreMesh(**_MESH),
        scratch_types=[
            pltpu.VMEM((ECH,), jnp.int32),
            pltpu.VMEM((ECH,), jnp.int32),
            pltpu.VMEM((ECH, 128), jnp.float32),
            pltpu.VMEM_SHARED((V, 128), jnp.float32),
            pltpu.SemaphoreType.DMA,
        ],
    )
    def g_kernel(x_hbm, src_hbm, dst_hbm, out_hbm, sidx_v, didx_v, rows_v,
                 acc_sh, sem):
        c = lax.axis_index("c")
        s = lax.axis_index("s")

        _zero_fill(rows_v, ECH, 128)
        n_full = VPT // ECH
        rem = VPT - n_full * ECH
        for j in range(n_full):
            pltpu.sync_copy(rows_v, acc_sh.at[pl.ds(s * VPT + j * ECH, ECH)])
        if rem:
            pltpu.sync_copy(rows_v.at[pl.ds(0, rem)],
                            acc_sh.at[pl.ds(s * VPT + n_full * ECH, rem)])
        plsc.subcore_barrier()

        def chunk(k, _):
            base = pl.multiple_of(c * EPC + s * EPT + k * ECH, 8)
            pltpu.sync_copy(src_hbm.at[pl.ds(base, ECH)], sidx_v)
            pltpu.sync_copy(dst_hbm.at[pl.ds(base, ECH)], didx_v)
            pltpu.async_copy(x_hbm.at[sidx_v], rows_v, sem).wait()
            pltpu.sync_copy(rows_v, acc_sh.at[didx_v], add=True)
            return 0

        lax.fori_loop(0, NCHUNK, chunk, 0)
        plsc.subcore_barrier()
        for j in range(n_full):
            pltpu.sync_copy(acc_sh.at[pl.ds(s * VPT + j * ECH, ECH)], rows_v)
            pltpu.sync_copy(rows_v, out_hbm.at[c, s, pl.ds(j * ECH, ECH)])
        if rem:
            pltpu.sync_copy(acc_sh.at[pl.ds(s * VPT + n_full * ECH, rem)],
                            rows_v.at[pl.ds(0, rem)])
            pltpu.sync_copy(rows_v.at[pl.ds(0, rem)],
                            out_hbm.at[c, s, pl.ds(n_full * ECH, rem)])

    return g_kernel


_G_RAW = _make_g_kernel()


def _G(xf, src, dst):
    p = _G_RAW(xf, src, dst).reshape(2, V, 128)
    return p[0], p[1]


def _row_spec(c):
    return pl.BlockSpec((RV, c), lambda i: (i, 0))


def _col_spec():
    return pl.BlockSpec((RV, 1), lambda i: (i, 0))


def _full_spec(shape):
    nd = len(shape)
    return pl.BlockSpec(shape, lambda i, _n=nd: (0,) * _n)


def _bd(w):
    z = jnp.zeros_like(w)
    return jnp.concatenate([jnp.concatenate([w, z], axis=1),
                            jnp.concatenate([z, w], axis=1)], axis=0)


def _pk(v):
    return jnp.concatenate([v, v], axis=1)


def _prep_body(gs0_ref, gs1_ref, gd0_ref, gd1_ref, a_ref, c_ref, m_ref):
    ds = gs0_ref[...][:, 0:1] + gs1_ref[...][:, 0:1]
    dd = gd0_ref[...][:, 0:1] + gd1_ref[...][:, 0:1]
    a = 1.0 / jnp.sqrt(jnp.maximum(ds, 1.0))
    c = 1.0 / jnp.sqrt(jnp.maximum(dd, 1.0))
    a_ref[...] = a
    c_ref[...] = c
    m_ref[...] = -(a * c)


def _prep(gs0, gs1, gd0, gd1):
    out = jax.ShapeDtypeStruct((V, 1), jnp.float32)
    return pl.pallas_call(
        _prep_body,
        grid=(GRID,),
        in_specs=[_row_spec(128)] * 4,
        out_specs=[_col_spec()] * 3,
        out_shape=[out, out, out],
    )(gs0, gs1, gd0, gd1)


def _scale_split_body(x_ref, s_ref, ya_ref, yb_ref):
    u = x_ref[...] * s_ref[...]
    ya_ref[...] = jnp.concatenate([u[:, 0:64], u[:, 128:192]], axis=1)
    yb_ref[...] = jnp.concatenate([u[:, 64:128], u[:, 192:256]], axis=1)


def _scale_split(x, s):
    out = jax.ShapeDtypeStruct((V, 128), jnp.float32)
    return pl.pallas_call(
        _scale_split_body,
        grid=(GRID,),
        in_specs=[_row_spec(256), _col_spec()],
        out_specs=[_row_spec(128)] * 2,
        out_shape=[out, out],
    )(x, s)


def _scale2_body(p0_ref, p1_ref, s_ref, y_ref):
    y_ref[...] = (p0_ref[...] + p1_ref[...]) * s_ref[...]


def _scale2(p0, p1, s):
    return pl.pallas_call(
        _scale2_body,
        grid=(GRID,),
        in_specs=[_row_spec(128), _row_spec(128), _col_spec()],
        out_specs=_row_spec(128),
        out_shape=jax.ShapeDtypeStruct((V, 128), jnp.float32),
    )(p0, p1, s)


def _combine_in_body(h_ref, g1a0_ref, g1a1_ref, g1b0_ref, g1b1_ref,
                     g2a0_ref, g2a1_ref, g2b0_ref, g2b1_ref, c_ref,
                     w_ref, b_ref, o_ref):
    c = c_ref[...]
    w1 = w_ref[1]
    w2 = w_ref[2]
    t1a = -(c * (g1a0_ref[...] + g1a1_ref[...]))
    t1b = -(c * (g1b0_ref[...] + g1b1_ref[...]))
    t2a = -2.0 * (c * (g2a0_ref[...] + g2a1_ref[...]))
    t2b = -2.0 * (c * (g2b0_ref[...] + g2b1_ref[...]))
    acc = jnp.dot(h_ref[...], _bd(w_ref[0] - w2),
                  preferred_element_type=jnp.float32, precision=_HI)
    acc = acc + jnp.dot(t1a, _bd(w1[:64]),
                        preferred_element_type=jnp.float32, precision=_HI)
    acc = acc + jnp.dot(t1b, _bd(w1[64:]),
                        preferred_element_type=jnp.float32, precision=_HI)
    acc = acc + jnp.dot(t2a, _bd(w2[:64]),
                        preferred_element_type=jnp.float32, precision=_HI)
    acc = acc + jnp.dot(t2b, _bd(w2[64:]),
                        preferred_element_type=jnp.float32, precision=_HI)
    acc = acc + _pk(b_ref[...])
    o_ref[...] = jnp.maximum(acc, 0.0)


def _combine_in(h, gs, c_row, W, b):
    return pl.pallas_call(
        _combine_in_body,
        grid=(GRID,),
        in_specs=[_row_spec(256)] + [_row_spec(128)] * 8 +
                 [_col_spec(), _full_spec((3, 128, 64)), _full_spec((1, 64))],
        out_specs=_row_spec(128),
        out_shape=jax.ShapeDtypeStruct((V, 128), jnp.float32),
    )(h, *gs, c_row, W, b.reshape(1, 64))


def _combine_body(relu, residual, cout, *refs):
    if residual:
        (h_ref, g10_ref, g11_ref, g20_ref, g21_ref, c_ref, w_ref, b_ref,
         r_ref, o_ref) = refs
    else:
        (h_ref, g10_ref, g11_ref, g20_ref, g21_ref, c_ref, w_ref, b_ref,
         o_ref) = refs
    c = c_ref[...]
    t1 = -(c * (g10_ref[...] + g11_ref[...]))
    t2 = -2.0 * (c * (g20_ref[...] + g21_ref[...]))
    acc = jnp.dot(h_ref[...], _bd(w_ref[0] - w_ref[2]),
                  preferred_element_type=jnp.float32, precision=_HI)
    acc = acc + jnp.dot(t1, _bd(w_ref[1]),
                        preferred_element_type=jnp.float32, precision=_HI)
    acc = acc + jnp.dot(t2, _bd(w_ref[2]),
                        preferred_element_type=jnp.float32, precision=_HI)
    acc = acc + _pk(b_ref[...])
    if residual:
        acc = acc + r_ref[...]
    if relu:
        acc = jnp.maximum(acc, 0.0)
    o_ref[...] = acc


def _combine(h, g10, g11, g20, g21, c_row, W, b, cout, relu, resid=None):
    ins = [h, g10, g11, g20, g21, c_row, W, b.reshape(1, cout)]
    specs = [_row_spec(128)] * 5 + [_col_spec(),
                                    _full_spec((3, 64, cout)),
                                    _full_spec((1, cout))]
    if resid is not None:
        ins.append(resid)
        specs.append(_row_spec(2 * cout))
    return pl.pallas_call(
        functools.partial(_combine_body, relu, resid is not None, cout),
        grid=(GRID,),
        in_specs=specs,
        out_specs=_row_spec(2 * cout),
        out_shape=jax.ShapeDtypeStruct((V, 2 * cout), jnp.float32),
    )(*ins)


def _stats_body(x_ref, s_ref, q_ref):
    @pl.when(pl.program_id(0) == 0)
    def _():
        s_ref[...] = jnp.zeros_like(s_ref)
        q_ref[...] = jnp.zeros_like(q_ref)

    x = x_ref[...]
    s_ref[...] += jnp.sum(x, axis=0, keepdims=True)
    q_ref[...] += jnp.sum(x * x, axis=0, keepdims=True)


def _bn_stats(x):
    out = jax.ShapeDtypeStruct((1, 128), jnp.float32)
    return pl.pallas_call(
        _stats_body,
        grid=(GRID,),
        in_specs=[_row_spec(128)],
        out_specs=[_full_spec((1, 128))] * 2,
        out_shape=[out, out],
    )(x)


def _bn_apply_body(x_ref, s_ref, q_ref, g_ref, be_ref, a_ref, h_ref, u_ref):
    s = s_ref[...]
    q = q_ref[...]
    n = float(B * V)
    mean = (s[:, :64] + s[:, 64:]) * (1.0 / n)
    var = (q[:, :64] + q[:, 64:]) * (1.0 / n) - mean * mean
    scale = g_ref[...] / jnp.sqrt(var + 1e-5)
    shift = be_ref[...] - mean * scale
    h = x_ref[...] * _pk(scale) + _pk(shift)
    h_ref[...] = h
    u_ref[...] = h * a_ref[...]


def _bn_apply(x, sums, sq, g, be, a_row):
    out = jax.ShapeDtypeStruct((V, 128), jnp.float32)
    return pl.pallas_call(
        _bn_apply_body,
        grid=(GRID,),
        in_specs=[_row_spec(128), _full_spec((1, 128)), _full_spec((1, 128)),
                  _full_spec((1, 64)), _full_spec((1, 64)), _col_spec()],
        out_specs=[_row_spec(128)] * 2,
        out_shape=[out, out],
    )(x, sums, sq, g.reshape(1, 64), be.reshape(1, 64), a_row)


def _pool_body(y_ref, o_ref):
    i = pl.program_id(0)

    @pl.when(i == 0)
    def _():
        o_ref[...] = jnp.full_like(o_ref, -jnp.inf)

    blk = jnp.max(y_ref[...], axis=0, keepdims=True)
    two = jnp.concatenate([blk[:, :10], blk[:, 10:]], axis=0)
    o_ref[...] = jnp.maximum(o_ref[...], two)

    @pl.when(i == GRID - 1)
    def _():
        v = o_ref[...]
        m = jnp.max(v, axis=1, keepdims=True)
        e = jnp.exp(v - m)
        o_ref[...] = v - m - jnp.log(jnp.sum(e, axis=1, keepdims=True))


def _pool(y):
    return pl.pallas_call(
        _pool_body,
        grid=(GRID,),
        in_specs=[_row_spec(20)],
        out_specs=pl.BlockSpec((B, 10), lambda i: (0, 0)),
        out_shape=jax.ShapeDtypeStruct((B, 10), jnp.float32),
    )(y)


def _cheb64(h, u0, src, dst, c_row, m_row, W, b, cout, relu, resid=None):
    g10, g11 = _G(u0, src, dst)
    u1 = _scale2(g10, g11, m_row)
    g20, g21 = _G(u1, src, dst)
    return _combine(h, g10, g11, g20, g21, c_row, W, b, cout, relu, resid)


def kernel(x, edge_index, W_in, b_in, g1, be1, W_r1, b_r1, g2, be2, W_r2,
           b_r2, g_out, be_out, W_out, b_out):
    ei = edge_index.astype(jnp.int32)
    src = ei[0]
    dst = ei[1]
    xt = jnp.transpose(x, (2, 0, 1)).reshape(V, 2 * 128)

    ones_t = jnp.ones((V, 128), jnp.float32)
    gd0, gd1 = _G(ones_t, src, dst)
    gs0, gs1 = _G(ones_t, dst, src)
    a_row, c_row, m_row = _prep(gs0, gs1, gd0, gd1)

    u0a, u0b = _scale_split(xt, a_row)
    g1a0, g1a1 = _G(u0a, src, dst)
    g1b0, g1b1 = _G(u0b, src, dst)
    u1a = _scale2(g1a0, g1a1, m_row)
    u1b = _scale2(g1b0, g1b1, m_row)
    g2a0, g2a1 = _G(u1a, src, dst)
    g2b0, g2b1 = _G(u1b, src, dst)
    h0 = _combine_in(xt, (g1a0, g1a1, g1b0, g1b1, g2a0, g2a1, g2b0, g2b1),
                     c_row, W_in, b_in)

    s1, q1 = _bn_stats(h0)
    hb, u0 = _bn_apply(h0, s1, q1, g1, be1, a_row)
    o = _cheb64(hb, u0, src, dst, c_row, m_row, W_r1, b_r1, 64, relu=True)
    s2, q2 = _bn_stats(o)
    ob, u0 = _bn_apply(o, s2, q2, g2, be2, a_row)
    out = _cheb64(ob, u0, src, dst, c_row, m_row, W_r2, b_r2, 64,
                  relu=True, resid=hb)

    s3, q3 = _bn_stats(out)
    z, u0 = _bn_apply(out, s3, q3, g_out, be_out, a_row)
    y = _cheb64(z, u0, src, dst, c_row, m_row, W_out, b_out, 10, relu=True)
    return _pool(y)


# speedup vs baseline: 6.7372x; 1.1850x over previous
import functools

import jax
import jax.numpy as jnp
from jax import lax
from jax.experimental import pallas as pl
from jax.experimental.pallas import tpu as pltpu
from jax.experimental.pallas import tpu_sc as plsc

V = 10000
E = 160000
B = 2
RV = 2000
GRID = V // RV

NS = 16
VPT = V // NS
EPC = E // 2
EPT = EPC // NS
ECH = 200

_MESH = dict(core_axis_name="c", subcore_axis_name="s")
_HI = lax.Precision.HIGHEST


def _zero_fill(ref, rows, cols):
    zv = jnp.zeros((16,), jnp.float32)

    def body(r, _):
        for j in range(cols // 16):
            ref[r, pl.ds(j * 16, 16)] = zv
        return 0

    lax.fori_loop(0, rows, body, 0)


def _make_g_kernel():
    NCHUNK = EPT // ECH

    @functools.partial(
        pl.kernel,
        out_type=jax.ShapeDtypeStruct((2, NS, VPT, 128), jnp.float32),
        mesh=plsc.VectorSubcoreMesh(**_MESH),
        scratch_types=[
            pltpu.VMEM((EPT,), jnp.int32),
            pltpu.VMEM((ECH,), jnp.int32),
            pltpu.VMEM((ECH,), jnp.int32),
            pltpu.VMEM((ECH, 128), jnp.float32),
            pltpu.VMEM_SHARED((V, 128), jnp.float32),
            pltpu.SemaphoreType.DMA,
            pltpu.SemaphoreType.DMA,
            pltpu.SemaphoreType.DMA,
        ],
    )
    def g_kernel(x_hbm, src_hbm, dst_hbm, out_hbm, sidx_v, d0, d1, rows_v,
                 acc_sh, sg, sd0, sd1):
        c = lax.axis_index("c")
        s = lax.axis_index("s")
        tbase = pl.multiple_of(c * EPC + s * EPT, 8)

        _zero_fill(rows_v, ECH, 128)
        n_full = VPT // ECH
        rem = VPT - n_full * ECH
        for j in range(n_full):
            pltpu.sync_copy(rows_v, acc_sh.at[pl.ds(s * VPT + j * ECH, ECH)])
        if rem:
            pltpu.sync_copy(rows_v.at[pl.ds(0, rem)],
                            acc_sh.at[pl.ds(s * VPT + n_full * ECH, rem)])
        pltpu.sync_copy(src_hbm.at[pl.ds(tbase, EPT)], sidx_v)
        plsc.subcore_barrier()

        ds_ = (d0, d1)
        sds = (sd0, sd1)

        def didx_start(k, p):
            return pltpu.async_copy(
                dst_hbm.at[pl.ds(tbase + k * ECH, ECH)], ds_[p], sds[p])

        def didx_wait(k, p):
            pltpu.make_async_copy(
                dst_hbm.at[pl.ds(tbase + k * ECH, ECH)], ds_[p],
                sds[p]).wait()

        def half(k, p):
            kb = pl.multiple_of(k * ECH, 8)
            gg = pltpu.async_copy(x_hbm.at[sidx_v.at[pl.ds(kb, ECH)]],
                                  rows_v, sg)
            didx_wait(k, p)
            if not isinstance(k, int) or k + 1 < NCHUNK:
                didx_start(k + 1, 1 - p)
            gg.wait()
            pltpu.sync_copy(rows_v, acc_sh.at[ds_[p]], add=True)

        didx_start(0, 0)

        def pair(i, _):
            half(i * 2, 0)
            half(i * 2 + 1, 1)
            return 0

        lax.fori_loop(0, NCHUNK // 2, pair, 0)
        half(NCHUNK - 1, 0)
        plsc.subcore_barrier()
        for j in range(n_full):
            pltpu.sync_copy(acc_sh.at[pl.ds(s * VPT + j * ECH, ECH)], rows_v)
            pltpu.sync_copy(rows_v, out_hbm.at[c, s, pl.ds(j * ECH, ECH)])
        if rem:
            pltpu.sync_copy(acc_sh.at[pl.ds(s * VPT + n_full * ECH, rem)],
                            rows_v.at[pl.ds(0, rem)])
            pltpu.sync_copy(rows_v.at[pl.ds(0, rem)],
                            out_hbm.at[c, s, pl.ds(n_full * ECH, rem)])

    return g_kernel


_G_RAW = _make_g_kernel()


def _G(xf, src, dst):
    p = _G_RAW(xf, src, dst).reshape(2, V, 128)
    return p[0], p[1]


def _row_spec(c):
    return pl.BlockSpec((RV, c), lambda i: (i, 0))


def _col_spec():
    return pl.BlockSpec((RV, 1), lambda i: (i, 0))


def _full_spec(shape):
    nd = len(shape)
    return pl.BlockSpec(shape, lambda i, _n=nd: (0,) * _n)


def _bd(w):
    z = jnp.zeros_like(w)
    return jnp.concatenate([jnp.concatenate([w, z], axis=1),
                            jnp.concatenate([z, w], axis=1)], axis=0)


def _pk(v):
    return jnp.concatenate([v, v], axis=1)


def _prep_body(gs0_ref, gs1_ref, gd0_ref, gd1_ref, a_ref, c_ref, m_ref):
    ds = gs0_ref[...][:, 0:1] + gs1_ref[...][:, 0:1]
    dd = gd0_ref[...][:, 0:1] + gd1_ref[...][:, 0:1]
    a = 1.0 / jnp.sqrt(jnp.maximum(ds, 1.0))
    c = 1.0 / jnp.sqrt(jnp.maximum(dd, 1.0))
    a_ref[...] = a
    c_ref[...] = c
    m_ref[...] = -(a * c)


def _prep(gs0, gs1, gd0, gd1):
    out = jax.ShapeDtypeStruct((V, 1), jnp.float32)
    return pl.pallas_call(
        _prep_body,
        grid=(GRID,),
        in_specs=[_row_spec(128)] * 4,
        out_specs=[_col_spec()] * 3,
        out_shape=[out, out, out],
    )(gs0, gs1, gd0, gd1)


def _scale_split_body(x_ref, s_ref, ya_ref, yb_ref):
    u = x_ref[...] * s_ref[...]
    ya_ref[...] = jnp.concatenate([u[:, 0:64], u[:, 128:192]], axis=1)
    yb_ref[...] = jnp.concatenate([u[:, 64:128], u[:, 192:256]], axis=1)


def _scale_split(x, s):
    out = jax.ShapeDtypeStruct((V, 128), jnp.float32)
    return pl.pallas_call(
        _scale_split_body,
        grid=(GRID,),
        in_specs=[_row_spec(256), _col_spec()],
        out_specs=[_row_spec(128)] * 2,
        out_shape=[out, out],
    )(x, s)


def _scale2_body(p0_ref, p1_ref, s_ref, y_ref):
    y_ref[...] = (p0_ref[...] + p1_ref[...]) * s_ref[...]


def _scale2(p0, p1, s):
    return pl.pallas_call(
        _scale2_body,
        grid=(GRID,),
        in_specs=[_row_spec(128), _row_spec(128), _col_spec()],
        out_specs=_row_spec(128),
        out_shape=jax.ShapeDtypeStruct((V, 128), jnp.float32),
    )(p0, p1, s)


def _combine_in_body(h_ref, g1a0_ref, g1a1_ref, g1b0_ref, g1b1_ref,
                     g2a0_ref, g2a1_ref, g2b0_ref, g2b1_ref, c_ref,
                     w_ref, b_ref, o_ref):
    c = c_ref[...]
    w1 = w_ref[1]
    w2 = w_ref[2]
    t1a = -(c * (g1a0_ref[...] + g1a1_ref[...]))
    t1b = -(c * (g1b0_ref[...] + g1b1_ref[...]))
    t2a = -2.0 * (c * (g2a0_ref[...] + g2a1_ref[...]))
    t2b = -2.0 * (c * (g2b0_ref[...] + g2b1_ref[...]))
    acc = jnp.dot(h_ref[...], _bd(w_ref[0] - w2),
                  preferred_element_type=jnp.float32, precision=_HI)
    acc = acc + jnp.dot(t1a, _bd(w1[:64]),
                        preferred_element_type=jnp.float32, precision=_HI)
    acc = acc + jnp.dot(t1b, _bd(w1[64:]),
                        preferred_element_type=jnp.float32, precision=_HI)
    acc = acc + jnp.dot(t2a, _bd(w2[:64]),
                        preferred_element_type=jnp.float32, precision=_HI)
    acc = acc + jnp.dot(t2b, _bd(w2[64:]),
                        preferred_element_type=jnp.float32, precision=_HI)
    acc = acc + _pk(b_ref[...])
    o_ref[...] = jnp.maximum(acc, 0.0)


def _combine_in(h, gs, c_row, W, b):
    return pl.pallas_call(
        _combine_in_body,
        grid=(GRID,),
        in_specs=[_row_spec(256)] + [_row_spec(128)] * 8 +
                 [_col_spec(), _full_spec((3, 128, 64)), _full_spec((1, 64))],
        out_specs=_row_spec(128),
        out_shape=jax.ShapeDtypeStruct((V, 128), jnp.float32),
    )(h, *gs, c_row, W, b.reshape(1, 64))


def _combine_body(relu, residual, cout, *refs):
    if residual:
        (h_ref, g10_ref, g11_ref, g20_ref, g21_ref, c_ref, w_ref, b_ref,
         r_ref, o_ref) = refs
    else:
        (h_ref, g10_ref, g11_ref, g20_ref, g21_ref, c_ref, w_ref, b_ref,
         o_ref) = refs
    c = c_ref[...]
    t1 = -(c * (g10_ref[...] + g11_ref[...]))
    t2 = -2.0 * (c * (g20_ref[...] + g21_ref[...]))
    acc = jnp.dot(h_ref[...], _bd(w_ref[0] - w_ref[2]),
                  preferred_element_type=jnp.float32, precision=_HI)
    acc = acc + jnp.dot(t1, _bd(w_ref[1]),
                        preferred_element_type=jnp.float32, precision=_HI)
    acc = acc + jnp.dot(t2, _bd(w_ref[2]),
                        preferred_element_type=jnp.float32, precision=_HI)
    acc = acc + _pk(b_ref[...])
    if residual:
        acc = acc + r_ref[...]
    if relu:
        acc = jnp.maximum(acc, 0.0)
    o_ref[...] = acc


def _combine(h, g10, g11, g20, g21, c_row, W, b, cout, relu, resid=None):
    ins = [h, g10, g11, g20, g21, c_row, W, b.reshape(1, cout)]
    specs = [_row_spec(128)] * 5 + [_col_spec(),
                                    _full_spec((3, 64, cout)),
                                    _full_spec((1, cout))]
    if resid is not None:
        ins.append(resid)
        specs.append(_row_spec(2 * cout))
    return pl.pallas_call(
        functools.partial(_combine_body, relu, resid is not None, cout),
        grid=(GRID,),
        in_specs=specs,
        out_specs=_row_spec(2 * cout),
        out_shape=jax.ShapeDtypeStruct((V, 2 * cout), jnp.float32),
    )(*ins)


def _stats_body(x_ref, s_ref, q_ref):
    @pl.when(pl.program_id(0) == 0)
    def _():
        s_ref[...] = jnp.zeros_like(s_ref)
        q_ref[...] = jnp.zeros_like(q_ref)

    x = x_ref[...]
    s_ref[...] += jnp.sum(x, axis=0, keepdims=True)
    q_ref[...] += jnp.sum(x * x, axis=0, keepdims=True)


def _bn_stats(x):
    out = jax.ShapeDtypeStruct((1, 128), jnp.float32)
    return pl.pallas_call(
        _stats_body,
        grid=(GRID,),
        in_specs=[_row_spec(128)],
        out_specs=[_full_spec((1, 128))] * 2,
        out_shape=[out, out],
    )(x)


def _bn_apply_body(x_ref, s_ref, q_ref, g_ref, be_ref, a_ref, h_ref, u_ref):
    s = s_ref[...]
    q = q_ref[...]
    n = float(B * V)
    mean = (s[:, :64] + s[:, 64:]) * (1.0 / n)
    var = (q[:, :64] + q[:, 64:]) * (1.0 / n) - mean * mean
    scale = g_ref[...] / jnp.sqrt(var + 1e-5)
    shift = be_ref[...] - mean * scale
    h = x_ref[...] * _pk(scale) + _pk(shift)
    h_ref[...] = h
    u_ref[...] = h * a_ref[...]


def _bn_apply(x, sums, sq, g, be, a_row):
    out = jax.ShapeDtypeStruct((V, 128), jnp.float32)
    return pl.pallas_call(
        _bn_apply_body,
        grid=(GRID,),
        in_specs=[_row_spec(128), _full_spec((1, 128)), _full_spec((1, 128)),
                  _full_spec((1, 64)), _full_spec((1, 64)), _col_spec()],
        out_specs=[_row_spec(128)] * 2,
        out_shape=[out, out],
    )(x, sums, sq, g.reshape(1, 64), be.reshape(1, 64), a_row)


def _pool_body(y_ref, o_ref):
    i = pl.program_id(0)

    @pl.when(i == 0)
    def _():
        o_ref[...] = jnp.full_like(o_ref, -jnp.inf)

    blk = jnp.max(y_ref[...], axis=0, keepdims=True)
    two = jnp.concatenate([blk[:, :10], blk[:, 10:]], axis=0)
    o_ref[...] = jnp.maximum(o_ref[...], two)

    @pl.when(i == GRID - 1)
    def _():
        v = o_ref[...]
        m = jnp.max(v, axis=1, keepdims=True)
        e = jnp.exp(v - m)
        o_ref[...] = v - m - jnp.log(jnp.sum(e, axis=1, keepdims=True))


def _pool(y):
    return pl.pallas_call(
        _pool_body,
        grid=(GRID,),
        in_specs=[_row_spec(20)],
        out_specs=pl.BlockSpec((B, 10), lambda i: (0, 0)),
        out_shape=jax.ShapeDtypeStruct((B, 10), jnp.float32),
    )(y)


def _cheb64(h, u0, src, dst, c_row, m_row, W, b, cout, relu, resid=None):
    g10, g11 = _G(u0, src, dst)
    u1 = _scale2(g10, g11, m_row)
    g20, g21 = _G(u1, src, dst)
    return _combine(h, g10, g11, g20, g21, c_row, W, b, cout, relu, resid)


def kernel(x, edge_index, W_in, b_in, g1, be1, W_r1, b_r1, g2, be2, W_r2,
           b_r2, g_out, be_out, W_out, b_out):
    ei = edge_index.astype(jnp.int32)
    src = ei[0]
    dst = ei[1]
    xt = jnp.transpose(x, (2, 0, 1)).reshape(V, 2 * 128)

    ones_t = jnp.ones((V, 128), jnp.float32)
    gd0, gd1 = _G(ones_t, src, dst)
    gs0, gs1 = _G(ones_t, dst, src)
    a_row, c_row, m_row = _prep(gs0, gs1, gd0, gd1)

    u0a, u0b = _scale_split(xt, a_row)
    g1a0, g1a1 = _G(u0a, src, dst)
    g1b0, g1b1 = _G(u0b, src, dst)
    u1a = _scale2(g1a0, g1a1, m_row)
    u1b = _scale2(g1b0, g1b1, m_row)
    g2a0, g2a1 = _G(u1a, src, dst)
    g2b0, g2b1 = _G(u1b, src, dst)
    h0 = _combine_in(xt, (g1a0, g1a1, g1b0, g1b1, g2a0, g2a1, g2b0, g2b1),
                     c_row, W_in, b_in)

    s1, q1 = _bn_stats(h0)
    hb, u0 = _bn_apply(h0, s1, q1, g1, be1, a_row)
    o = _cheb64(hb, u0, src, dst, c_row, m_row, W_r1, b_r1, 64, relu=True)
    s2, q2 = _bn_stats(o)
    ob, u0 = _bn_apply(o, s2, q2, g2, be2, a_row)
    out = _cheb64(ob, u0, src, dst, c_row, m_row, W_r2, b_r2, 64,
                  relu=True, resid=hb)

    s3, q3 = _bn_stats(out)
    z, u0 = _bn_apply(out, s3, q3, g_out, be_out, a_row)
    y = _cheb64(z, u0, src, dst, c_row, m_row, W_out, b_out, 10, relu=True)
    return _pool(y)
